# baseline (speedup 1.0000x reference)
import functools

import jax
import jax.numpy as jnp
from jax import lax
from jax.experimental import pallas as pl
from jax.experimental.pallas import tpu as pltpu
from jax.experimental.pallas import tpu_sc as plsc

B = 16
L = 128
V = 32000
E = 512
H = 512
NF = 256
FR = 192
FE = 128
P = NF - FR

TW = 32
NBLK = L // TW


def _dotT(a, b):
    return lax.dot_general(
        a, b, (((1,), (1,)), ((), ())), preferred_element_type=jnp.float32
    )


def _gather_rows_sc(table, idx):
    n = idx.shape[0]
    d = table.shape[1]
    mesh = plsc.VectorSubcoreMesh(core_axis_name="c", subcore_axis_name="s")
    nw = mesh.num_cores * mesh.num_subcores
    b_per_w = n // nw

    @functools.partial(
        pl.kernel,
        out_type=jax.ShapeDtypeStruct((n, d), jnp.float32),
        mesh=mesh,
        scratch_types=[
            pltpu.VMEM((b_per_w,), jnp.int32),
            pltpu.VMEM((b_per_w, d), jnp.float32),
            pltpu.SemaphoreType.DMA,
        ],
    )
    def gather_kernel(table_hbm, idx_hbm, out_hbm, idx_v, rows_v, sem):
        wid = lax.axis_index("s") * mesh.num_cores + lax.axis_index("c")
        base = wid * b_per_w
        pltpu.sync_copy(idx_hbm.at[pl.ds(base, b_per_w)], idx_v)
        pltpu.async_copy(table_hbm.at[idx_v], rows_v, sem).wait()
        pltpu.sync_copy(rows_v, out_hbm.at[pl.ds(base, b_per_w)])

    return gather_kernel(table, idx)


def _gi0_kernel(x_ref, wf_ref, wb_ref, bf_ref, bb_ref, gf_ref, gb_ref):
    x = x_ref[...].astype(jnp.bfloat16)
    rows = x.shape[0]
    gf = _dotT(x, wf_ref[...]) + bf_ref[...]
    gb = _dotT(x, wb_ref[...]) + bb_ref[...]
    gf_ref[...] = gf.reshape(rows // B, B, 3 * H).astype(jnp.bfloat16)
    gb_ref[...] = gb.reshape(rows // B, B, 3 * H).astype(jnp.bfloat16)


def _gi0_proj(x_flat, w_f, w_b, b_f, b_b):
    tl = 64
    rows = tl * B
    out_shape = jax.ShapeDtypeStruct((L, B, 3 * H), jnp.bfloat16)
    return pl.pallas_call(
        _gi0_kernel,
        grid=(L // tl,),
        in_specs=[
            pl.BlockSpec((rows, E), lambda i: (i, 0)),
            pl.BlockSpec((3 * H, E), lambda i: (0, 0)),
            pl.BlockSpec((3 * H, E), lambda i: (0, 0)),
            pl.BlockSpec((1, 3 * H), lambda i: (0, 0)),
            pl.BlockSpec((1, 3 * H), lambda i: (0, 0)),
        ],
        out_specs=[
            pl.BlockSpec((tl, B, 3 * H), lambda i: (i, 0, 0)),
            pl.BlockSpec((tl, B, 3 * H), lambda i: (i, 0, 0)),
        ],
        out_shape=[out_shape, out_shape],
    )(x_flat, w_f, w_b, b_f.reshape(1, 3 * H), b_b.reshape(1, 3 * H))


def _gi1_kernel(yf_ref, yb_ref, wf_ref, wb_ref, bf_ref, bb_ref, gf_ref, gb_ref):
    yf = yf_ref[...]
    yb = yb_ref[...]
    rows = yf.shape[0]
    wf = wf_ref[...]
    wb = wb_ref[...]
    gf = _dotT(yf, wf[:, :H]) + _dotT(yb, wf[:, H:]) + bf_ref[...]
    gb = _dotT(yf, wb[:, :H]) + _dotT(yb, wb[:, H:]) + bb_ref[...]
    gf_ref[...] = gf.reshape(rows // B, B, 3 * H).astype(jnp.bfloat16)
    gb_ref[...] = gb.reshape(rows // B, B, 3 * H).astype(jnp.bfloat16)


def _gi1_proj(yf_flat, yb_flat, w_f, w_b, b_f, b_b):
    tl = 64
    rows = tl * B
    out_shape = jax.ShapeDtypeStruct((L, B, 3 * H), jnp.bfloat16)
    return pl.pallas_call(
        _gi1_kernel,
        grid=(L // tl,),
        in_specs=[
            pl.BlockSpec((rows, H), lambda i: (i, 0)),
            pl.BlockSpec((rows, H), lambda i: (i, 0)),
            pl.BlockSpec((3 * H, 2 * H), lambda i: (0, 0)),
            pl.BlockSpec((3 * H, 2 * H), lambda i: (0, 0)),
            pl.BlockSpec((1, 3 * H), lambda i: (0, 0)),
            pl.BlockSpec((1, 3 * H), lambda i: (0, 0)),
        ],
        out_specs=[
            pl.BlockSpec((tl, B, 3 * H), lambda i: (i, 0, 0)),
            pl.BlockSpec((tl, B, 3 * H), lambda i: (i, 0, 0)),
        ],
        out_shape=[out_shape, out_shape],
    )(yf_flat, yb_flat, w_f, w_b, b_f.reshape(1, 3 * H), b_b.reshape(1, 3 * H))


def _gru_cell(gi, gh, h):
    ir, iz, inn = gi[:, :H], gi[:, H : 2 * H], gi[:, 2 * H :]
    hr, hz, hn = gh[:, :H], gh[:, H : 2 * H], gh[:, 2 * H :]
    r = jax.nn.sigmoid(ir + hr)
    z = jax.nn.sigmoid(iz + hz)
    n = jnp.tanh(inn + r * hn)
    return (1.0 - z) * n + z * h


def _cell_step(gi, h, wh, bh):
    gh = lax.dot_general(
        h.astype(jnp.bfloat16), wh, (((1,), (0,)), ((), ())),
        preferred_element_type=jnp.float32,
    ) + bh
    return _gru_cell(gi, gh, h)


def _gru_scan_kernel_t(
    gif_ref, gib_ref, whf_ref, whb_ref, bhf_ref, bhb_ref, h0f_ref, h0b_ref,
    yf_ref, yb_ref, hf, hb, wtf, wtb,
):
    i = pl.program_id(0)

    @pl.when(i == 0)
    def _():
        hf[...] = h0f_ref[...]
        hb[...] = h0b_ref[...]

    @pl.when(i == 0)
    def _():
        wtf[...] = whf_ref[...].T.astype(jnp.bfloat16)
        wtb[...] = whb_ref[...].T.astype(jnp.bfloat16)

    whf = wtf[...]
    whb = wtb[...]
    bhf = bhf_ref[...]
    bhb = bhb_ref[...]
    h_f = hf[...]
    h_b = hb[...]
    for k in range(TW):
        h_f = _cell_step(gif_ref[k], h_f, whf, bhf)
        yf_ref[k] = h_f.astype(jnp.bfloat16)
        h_b = _cell_step(gib_ref[TW - 1 - k], h_b, whb, bhb)
        yb_ref[TW - 1 - k] = h_b.astype(jnp.bfloat16)
    hf[...] = h_f
    hb[...] = h_b


def _gru_scan_kernel_b(
    gif_ref, gib_ref, whf_ref, whb_ref, bhf_ref, bhb_ref, h0f_ref, h0b_ref,
    yf_ref, yb_ref, hf, hb, wtf, wtb,
):
    i = pl.program_id(0)

    @pl.when(i == 0)
    def _():
        hf[...] = h0f_ref[...]
        hb[...] = h0b_ref[...]

    @pl.when(i == 0)
    def _():
        wtf[...] = whf_ref[...].T.astype(jnp.bfloat16)
        wtb[...] = whb_ref[...].T.astype(jnp.bfloat16)

    whf = wtf[...]
    whb = wtb[...]
    bhf = bhf_ref[...]
    bhb = bhb_ref[...]
    h_f = hf[...]
    h_b = hb[...]
    for k in range(TW):
        h_f = _cell_step(gif_ref[k], h_f, whf, bhf)
        yf_ref[:, k, :] = h_f
        h_b = _cell_step(gib_ref[TW - 1 - k], h_b, whb, bhb)
        yb_ref[:, TW - 1 - k, :] = h_b
    hf[...] = h_f
    hb[...] = h_b


def _gru_bidir(gi_f, gi_b, w_hh_f, w_hh_b, b_hh_f, b_hh_b, h0f, h0b, bmajor):
    if bmajor:
        body = _gru_scan_kernel_b
        out_shape = jax.ShapeDtypeStruct((B, L, H), jnp.float32)
        out_specs = [
            pl.BlockSpec((B, TW, H), lambda i: (0, i, 0)),
            pl.BlockSpec((B, TW, H), lambda i: (0, NBLK - 1 - i, 0)),
        ]
    else:
        body = _gru_scan_kernel_t
        out_shape = jax.ShapeDtypeStruct((L, B, H), jnp.bfloat16)
        out_specs = [
            pl.BlockSpec((TW, B, H), lambda i: (i, 0, 0)),
            pl.BlockSpec((TW, B, H), lambda i: (NBLK - 1 - i, 0, 0)),
        ]
    return pl.pallas_call(
        body,
        grid=(NBLK,),
        in_specs=[
            pl.BlockSpec((TW, B, 3 * H), lambda i: (i, 0, 0)),
            pl.BlockSpec((TW, B, 3 * H), lambda i: (NBLK - 1 - i, 0, 0)),
            pl.BlockSpec((3 * H, H), lambda i: (0, 0)),
            pl.BlockSpec((3 * H, H), lambda i: (0, 0)),
            pl.BlockSpec((1, 3 * H), lambda i: (0, 0)),
            pl.BlockSpec((1, 3 * H), lambda i: (0, 0)),
            pl.BlockSpec((B, H), lambda i: (0, 0)),
            pl.BlockSpec((B, H), lambda i: (0, 0)),
        ],
        out_specs=out_specs,
        out_shape=[out_shape, out_shape],
        scratch_shapes=[
            pltpu.VMEM((B, H), jnp.float32),
            pltpu.VMEM((B, H), jnp.float32),
            pltpu.VMEM((H, 3 * H), jnp.bfloat16),
            pltpu.VMEM((H, 3 * H), jnp.bfloat16),
        ],
    )(
        gi_f, gi_b,
        w_hh_f, w_hh_b,
        b_hh_f.reshape(1, 3 * H), b_hh_b.reshape(1, 3 * H), h0f, h0b,
    )


def _energy_softmax_kernel(
    yf_ref, yb_ref, fe_ref, fw_ref, fb_ref, v_ref, lw_ref, lb_ref, soft_ref
):
    fe = fe_ref[...]
    vv = v_ref[...]
    mf = jnp.dot(fe, fw_ref[...], preferred_element_type=jnp.float32)
    mv = jnp.dot(vv, lw_ref[...], preferred_element_type=jnp.float32)
    m_mat = jnp.concatenate([mf, mv], axis=0)
    cf = _dotT(fb_ref[...], fe)
    cv = _dotT(lb_ref[...], vv)
    c = jnp.concatenate([cf, cv], axis=1)
    energy = _dotT(yf_ref[...], m_mat[:, :H]) + _dotT(yb_ref[...], m_mat[:, H:]) + c
    e3 = energy.reshape(B, L, NF)
    mx = jnp.max(e3, axis=0, keepdims=True)
    ex = jnp.exp(e3 - mx)
    sm = jnp.sum(ex, axis=0, keepdims=True)
    soft_ref[...] = ex / sm


def _energy_softmax(yf_flat, yb_flat, feature_embeddings, feat_W, feat_b, v,
                    lin_W, lin_b):
    return pl.pallas_call(
        _energy_softmax_kernel,
        out_shape=jax.ShapeDtypeStruct((B, L, NF), jnp.float32),
    )(
        yf_flat, yb_flat, feature_embeddings, feat_W, feat_b.reshape(1, FE),
        v, lin_W, lin_b.reshape(1, H),
    )


def _context_kernel(soft_ref, yf_ref, yb_ref, ctx_ref, attn_ref):
    s = soft_ref[0]
    ctx_ref[0, :, :H] = lax.dot_general(
        s, yf_ref[0], (((0,), (0,)), ((), ())), preferred_element_type=jnp.float32
    )
    ctx_ref[0, :, H:] = lax.dot_general(
        s, yb_ref[0], (((0,), (0,)), ((), ())), preferred_element_type=jnp.float32
    )
    attn_ref[0] = s.T


def _context(soft, yf2, yb2):
    return pl.pallas_call(
        _context_kernel,
        grid=(B,),
        in_specs=[
            pl.BlockSpec((1, L, NF), lambda b: (b, 0, 0)),
            pl.BlockSpec((1, L, H), lambda b: (b, 0, 0)),
            pl.BlockSpec((1, L, H), lambda b: (b, 0, 0)),
        ],
        out_specs=[
            pl.BlockSpec((1, NF, 2 * H), lambda b: (b, 0, 0)),
            pl.BlockSpec((1, NF, L), lambda b: (b, 0, 0)),
        ],
        out_shape=[
            jax.ShapeDtypeStruct((B, NF, 2 * H), jnp.float32),
            jax.ShapeDtypeStruct((B, NF, L), jnp.float32),
        ],
    )(soft, yf2, yb2)


def _encode_tc(
    xs_flat, hidden,
    w_ih_l0f, w_hh_l0f, b_ih_l0f, b_hh_l0f,
    w_ih_l0b, w_hh_l0b, b_ih_l0b, b_hh_l0b,
    w_ih_l1f, w_hh_l1f, b_ih_l1f, b_hh_l1f,
    w_ih_l1b, w_hh_l1b, b_ih_l1b, b_hh_l1b,
    lin_W, lin_b, feat_W, feat_b, v, feature_embeddings,
):
    gi0f, gi0b = _gi0_proj(
        xs_flat,
        w_ih_l0f.astype(jnp.bfloat16), w_ih_l0b.astype(jnp.bfloat16),
        b_ih_l0f, b_ih_l0b,
    )
    yf, yb = _gru_bidir(
        gi0f, gi0b, w_hh_l0f, w_hh_l0b, b_hh_l0f, b_hh_l0b,
        hidden[0], hidden[1], bmajor=False,
    )
    gi1f, gi1b = _gi1_proj(
        yf.reshape(L * B, H), yb.reshape(L * B, H),
        w_ih_l1f.astype(jnp.bfloat16), w_ih_l1b.astype(jnp.bfloat16),
        b_ih_l1f, b_ih_l1b,
    )
    yf2, yb2 = _gru_bidir(
        gi1f, gi1b, w_hh_l1f, w_hh_l1b, b_hh_l1f, b_hh_l1b,
        hidden[2], hidden[3], bmajor=True,
    )
    soft = _energy_softmax(
        yf2.reshape(B * L, H), yb2.reshape(B * L, H),
        feature_embeddings, feat_W, feat_b, v, lin_W, lin_b,
    )
    context, attn_w = _context(soft, yf2, yb2)
    return context, attn_w


def kernel(
    input_variable, input_lengths, hidden, feature_embeddings, emb_table,
    w_ih_l0f, w_hh_l0f, b_ih_l0f, b_hh_l0f,
    w_ih_l0b, w_hh_l0b, b_ih_l0b, b_hh_l0b,
    w_ih_l1f, w_hh_l1f, b_ih_l1f, b_hh_l1f,
    w_ih_l1b, w_hh_l1b, b_ih_l1b, b_hh_l1b,
    lin_W, lin_b, feat_W, feat_b, v,
):
    idx_t = jnp.transpose(input_variable, (1, 0)).reshape(L * B).astype(jnp.int32)
    xs_flat = _gather_rows_sc(emb_table, idx_t)
    return _encode_tc(
        xs_flat, hidden,
        w_ih_l0f, w_hh_l0f, b_ih_l0f, b_hh_l0f,
        w_ih_l0b, w_hh_l0b, b_ih_l0b, b_hh_l0b,
        w_ih_l1f, w_hh_l1f, b_ih_l1f, b_hh_l1f,
        w_ih_l1b, w_hh_l1b, b_ih_l1b, b_hh_l1b,
        lin_W, lin_b, feat_W, feat_b, v, feature_embeddings,
    )


# speedup vs baseline: 1.0820x; 1.0820x over previous
import functools

import jax
import jax.numpy as jnp
from jax import lax
from jax.experimental import pallas as pl
from jax.experimental.pallas import tpu as pltpu
from jax.experimental.pallas import tpu_sc as plsc

B = 16
L = 128
V = 32000
E = 512
H = 512
NF = 256
FR = 192
FE = 128
P = NF - FR

TW = 16
NBLK = L // TW


def _dotT(a, b):
    return lax.dot_general(
        a, b, (((1,), (1,)), ((), ())), preferred_element_type=jnp.float32
    )


def _gather_rows_sc(table, idx):
    n = idx.shape[0]
    d = table.shape[1]
    mesh = plsc.VectorSubcoreMesh(core_axis_name="c", subcore_axis_name="s")
    nw = mesh.num_cores * mesh.num_subcores
    b_per_w = n // nw

    @functools.partial(
        pl.kernel,
        out_type=jax.ShapeDtypeStruct((n, d), jnp.float32),
        mesh=mesh,
        scratch_types=[
            pltpu.VMEM((b_per_w,), jnp.int32),
            pltpu.VMEM((b_per_w, d), jnp.float32),
            pltpu.SemaphoreType.DMA,
        ],
    )
    def gather_kernel(table_hbm, idx_hbm, out_hbm, idx_v, rows_v, sem):
        wid = lax.axis_index("s") * mesh.num_cores + lax.axis_index("c")
        base = wid * b_per_w
        pltpu.sync_copy(idx_hbm.at[pl.ds(base, b_per_w)], idx_v)
        pltpu.async_copy(table_hbm.at[idx_v], rows_v, sem).wait()
        pltpu.sync_copy(rows_v, out_hbm.at[pl.ds(base, b_per_w)])

    return gather_kernel(table, idx)


def _gi0_kernel(x_ref, wf_ref, wb_ref, bf_ref, bb_ref, gf_ref, gb_ref):
    x = x_ref[...].astype(jnp.bfloat16)
    rows = x.shape[0]
    gf = _dotT(x, wf_ref[...]) + bf_ref[...]
    gb = _dotT(x, wb_ref[...]) + bb_ref[...]
    gf_ref[...] = gf.reshape(rows // B, B, 3 * H).astype(jnp.bfloat16)
    gb_ref[...] = gb.reshape(rows // B, B, 3 * H).astype(jnp.bfloat16)


def _gi0_proj(x_flat, w_f, w_b, b_f, b_b):
    tl = 64
    rows = tl * B
    out_shape = jax.ShapeDtypeStruct((L, B, 3 * H), jnp.bfloat16)
    return pl.pallas_call(
        _gi0_kernel,
        grid=(L // tl,),
        in_specs=[
            pl.BlockSpec((rows, E), lambda i: (i, 0)),
            pl.BlockSpec((3 * H, E), lambda i: (0, 0)),
            pl.BlockSpec((3 * H, E), lambda i: (0, 0)),
            pl.BlockSpec((1, 3 * H), lambda i: (0, 0)),
            pl.BlockSpec((1, 3 * H), lambda i: (0, 0)),
        ],
        out_specs=[
            pl.BlockSpec((tl, B, 3 * H), lambda i: (i, 0, 0)),
            pl.BlockSpec((tl, B, 3 * H), lambda i: (i, 0, 0)),
        ],
        out_shape=[out_shape, out_shape],
    )(x_flat, w_f, w_b, b_f.reshape(1, 3 * H), b_b.reshape(1, 3 * H))


def _gi1_kernel(yf_ref, yb_ref, wf_ref, wb_ref, bf_ref, bb_ref, gf_ref, gb_ref):
    yf = yf_ref[...]
    yb = yb_ref[...]
    rows = yf.shape[0]
    wf = wf_ref[...]
    wb = wb_ref[...]
    gf = _dotT(yf, wf[:, :H]) + _dotT(yb, wf[:, H:]) + bf_ref[...]
    gb = _dotT(yf, wb[:, :H]) + _dotT(yb, wb[:, H:]) + bb_ref[...]
    gf_ref[...] = gf.reshape(rows // B, B, 3 * H).astype(jnp.bfloat16)
    gb_ref[...] = gb.reshape(rows // B, B, 3 * H).astype(jnp.bfloat16)


def _gi1_proj(yf_flat, yb_flat, w_f, w_b, b_f, b_b):
    tl = 64
    rows = tl * B
    out_shape = jax.ShapeDtypeStruct((L, B, 3 * H), jnp.bfloat16)
    return pl.pallas_call(
        _gi1_kernel,
        grid=(L // tl,),
        in_specs=[
            pl.BlockSpec((rows, H), lambda i: (i, 0)),
            pl.BlockSpec((rows, H), lambda i: (i, 0)),
            pl.BlockSpec((3 * H, 2 * H), lambda i: (0, 0)),
            pl.BlockSpec((3 * H, 2 * H), lambda i: (0, 0)),
            pl.BlockSpec((1, 3 * H), lambda i: (0, 0)),
            pl.BlockSpec((1, 3 * H), lambda i: (0, 0)),
        ],
        out_specs=[
            pl.BlockSpec((tl, B, 3 * H), lambda i: (i, 0, 0)),
            pl.BlockSpec((tl, B, 3 * H), lambda i: (i, 0, 0)),
        ],
        out_shape=[out_shape, out_shape],
    )(yf_flat, yb_flat, w_f, w_b, b_f.reshape(1, 3 * H), b_b.reshape(1, 3 * H))


def _gru_cell(gi, gh, h):
    ir, iz, inn = gi[:, :H], gi[:, H : 2 * H], gi[:, 2 * H :]
    hr, hz, hn = gh[:, :H], gh[:, H : 2 * H], gh[:, 2 * H :]
    r = jax.nn.sigmoid(ir + hr)
    z = jax.nn.sigmoid(iz + hz)
    n = jnp.tanh(inn + r * hn)
    return (1.0 - z) * n + z * h


def _cell_step(gi, h, wh, bh):
    gh = lax.dot_general(
        h.astype(jnp.bfloat16), wh, (((1,), (0,)), ((), ())),
        preferred_element_type=jnp.float32,
    ) + bh
    return _gru_cell(gi, gh, h)


def _gru_scan_kernel_t(
    gif_ref, gib_ref, whf_ref, whb_ref, bhf_ref, bhb_ref, h0f_ref, h0b_ref,
    yf_ref, yb_ref, hf, hb, wtf, wtb,
):
    i = pl.program_id(0)

    @pl.when(i == 0)
    def _():
        hf[...] = h0f_ref[...]
        hb[...] = h0b_ref[...]

    @pl.when(i == 0)
    def _():
        wtf[...] = whf_ref[...].T.astype(jnp.bfloat16)
        wtb[...] = whb_ref[...].T.astype(jnp.bfloat16)

    whf = wtf[...]
    whb = wtb[...]
    bhf = bhf_ref[...]
    bhb = bhb_ref[...]
    h_f = hf[...]
    h_b = hb[...]
    for k in range(TW):
        h_f = _cell_step(gif_ref[k], h_f, whf, bhf)
        yf_ref[k] = h_f.astype(jnp.bfloat16)
        h_b = _cell_step(gib_ref[TW - 1 - k], h_b, whb, bhb)
        yb_ref[TW - 1 - k] = h_b.astype(jnp.bfloat16)
    hf[...] = h_f
    hb[...] = h_b


def _gru_scan_kernel_b(
    gif_ref, gib_ref, whf_ref, whb_ref, bhf_ref, bhb_ref, h0f_ref, h0b_ref,
    yf_ref, yb_ref, hf, hb, wtf, wtb,
):
    i = pl.program_id(0)

    @pl.when(i == 0)
    def _():
        hf[...] = h0f_ref[...]
        hb[...] = h0b_ref[...]

    @pl.when(i == 0)
    def _():
        wtf[...] = whf_ref[...].T.astype(jnp.bfloat16)
        wtb[...] = whb_ref[...].T.astype(jnp.bfloat16)

    whf = wtf[...]
    whb = wtb[...]
    bhf = bhf_ref[...]
    bhb = bhb_ref[...]
    h_f = hf[...]
    h_b = hb[...]
    for k in range(TW):
        h_f = _cell_step(gif_ref[k], h_f, whf, bhf)
        yf_ref[:, k, :] = h_f
        h_b = _cell_step(gib_ref[TW - 1 - k], h_b, whb, bhb)
        yb_ref[:, TW - 1 - k, :] = h_b
    hf[...] = h_f
    hb[...] = h_b


def _gru_bidir(gi_f, gi_b, w_hh_f, w_hh_b, b_hh_f, b_hh_b, h0f, h0b, bmajor):
    if bmajor:
        body = _gru_scan_kernel_b
        out_shape = jax.ShapeDtypeStruct((B, L, H), jnp.float32)
        out_specs = [
            pl.BlockSpec((B, TW, H), lambda i: (0, i, 0)),
            pl.BlockSpec((B, TW, H), lambda i: (0, NBLK - 1 - i, 0)),
        ]
    else:
        body = _gru_scan_kernel_t
        out_shape = jax.ShapeDtypeStruct((L, B, H), jnp.bfloat16)
        out_specs = [
            pl.BlockSpec((TW, B, H), lambda i: (i, 0, 0)),
            pl.BlockSpec((TW, B, H), lambda i: (NBLK - 1 - i, 0, 0)),
        ]
    return pl.pallas_call(
        body,
        grid=(NBLK,),
        in_specs=[
            pl.BlockSpec((TW, B, 3 * H), lambda i: (i, 0, 0)),
            pl.BlockSpec((TW, B, 3 * H), lambda i: (NBLK - 1 - i, 0, 0)),
            pl.BlockSpec((3 * H, H), lambda i: (0, 0)),
            pl.BlockSpec((3 * H, H), lambda i: (0, 0)),
            pl.BlockSpec((1, 3 * H), lambda i: (0, 0)),
            pl.BlockSpec((1, 3 * H), lambda i: (0, 0)),
            pl.BlockSpec((B, H), lambda i: (0, 0)),
            pl.BlockSpec((B, H), lambda i: (0, 0)),
        ],
        out_specs=out_specs,
        out_shape=[out_shape, out_shape],
        scratch_shapes=[
            pltpu.VMEM((B, H), jnp.float32),
            pltpu.VMEM((B, H), jnp.float32),
            pltpu.VMEM((H, 3 * H), jnp.bfloat16),
            pltpu.VMEM((H, 3 * H), jnp.bfloat16),
        ],
    )(
        gi_f, gi_b,
        w_hh_f, w_hh_b,
        b_hh_f.reshape(1, 3 * H), b_hh_b.reshape(1, 3 * H), h0f, h0b,
    )


def _attn_kernel(
    yf_ref, yb_ref, fe_ref, fw_ref, fb_ref, v_ref, lw_ref, lb_ref,
    ctx_ref, attn_ref,
):
    fe = fe_ref[...]
    vv = v_ref[...]
    mf = jnp.dot(fe, fw_ref[...], preferred_element_type=jnp.float32)
    mv = jnp.dot(vv, lw_ref[...], preferred_element_type=jnp.float32)
    m_mat = jnp.concatenate([mf, mv], axis=0)
    cf = _dotT(fb_ref[...], fe)
    cv = _dotT(lb_ref[...], vv)
    c = jnp.concatenate([cf, cv], axis=1)
    yf = yf_ref[...]
    yb = yb_ref[...]
    energy = (
        _dotT(yf.reshape(B * L, H), m_mat[:, :H])
        + _dotT(yb.reshape(B * L, H), m_mat[:, H:])
        + c
    )
    e3 = energy.reshape(B, L, NF)
    mx = jnp.max(e3, axis=0, keepdims=True)
    ex = jnp.exp(e3 - mx)
    sm = jnp.sum(ex, axis=0, keepdims=True)
    soft3 = ex / sm
    for b in range(B):
        s = soft3[b]
        ctx_ref[b, :, :H] = lax.dot_general(
            s, yf[b], (((0,), (0,)), ((), ())),
            preferred_element_type=jnp.float32,
        )
        ctx_ref[b, :, H:] = lax.dot_general(
            s, yb[b], (((0,), (0,)), ((), ())),
            preferred_element_type=jnp.float32,
        )
        attn_ref[b] = s.T


def _attention(yf2, yb2, feature_embeddings, feat_W, feat_b, v, lin_W, lin_b):
    return pl.pallas_call(
        _attn_kernel,
        out_shape=[
            jax.ShapeDtypeStruct((B, NF, 2 * H), jnp.float32),
            jax.ShapeDtypeStruct((B, NF, L), jnp.float32),
        ],
    )(
        yf2, yb2, feature_embeddings, feat_W, feat_b.reshape(1, FE),
        v, lin_W, lin_b.reshape(1, H),
    )


def _encode_tc(
    xs_flat, hidden,
    w_ih_l0f, w_hh_l0f, b_ih_l0f, b_hh_l0f,
    w_ih_l0b, w_hh_l0b, b_ih_l0b, b_hh_l0b,
    w_ih_l1f, w_hh_l1f, b_ih_l1f, b_hh_l1f,
    w_ih_l1b, w_hh_l1b, b_ih_l1b, b_hh_l1b,
    lin_W, lin_b, feat_W, feat_b, v, feature_embeddings,
):
    gi0f, gi0b = _gi0_proj(
        xs_flat,
        w_ih_l0f.astype(jnp.bfloat16), w_ih_l0b.astype(jnp.bfloat16),
        b_ih_l0f, b_ih_l0b,
    )
    yf, yb = _gru_bidir(
        gi0f, gi0b, w_hh_l0f, w_hh_l0b, b_hh_l0f, b_hh_l0b,
        hidden[0], hidden[1], bmajor=False,
    )
    gi1f, gi1b = _gi1_proj(
        yf.reshape(L * B, H), yb.reshape(L * B, H),
        w_ih_l1f.astype(jnp.bfloat16), w_ih_l1b.astype(jnp.bfloat16),
        b_ih_l1f, b_ih_l1b,
    )
    yf2, yb2 = _gru_bidir(
        gi1f, gi1b, w_hh_l1f, w_hh_l1b, b_hh_l1f, b_hh_l1b,
        hidden[2], hidden[3], bmajor=True,
    )
    context, attn_w = _attention(
        yf2, yb2, feature_embeddings, feat_W, feat_b, v, lin_W, lin_b,
    )
    return context, attn_w


def kernel(
    input_variable, input_lengths, hidden, feature_embeddings, emb_table,
    w_ih_l0f, w_hh_l0f, b_ih_l0f, b_hh_l0f,
    w_ih_l0b, w_hh_l0b, b_ih_l0b, b_hh_l0b,
    w_ih_l1f, w_hh_l1f, b_ih_l1f, b_hh_l1f,
    w_ih_l1b, w_hh_l1b, b_ih_l1b, b_hh_l1b,
    lin_W, lin_b, feat_W, feat_b, v,
):
    idx_t = jnp.transpose(input_variable, (1, 0)).reshape(L * B).astype(jnp.int32)
    xs_flat = _gather_rows_sc(emb_table, idx_t)
    return _encode_tc(
        xs_flat, hidden,
        w_ih_l0f, w_hh_l0f, b_ih_l0f, b_hh_l0f,
        w_ih_l0b, w_hh_l0b, b_ih_l0b, b_hh_l0b,
        w_ih_l1f, w_hh_l1f, b_ih_l1f, b_hh_l1f,
        w_ih_l1b, w_hh_l1b, b_ih_l1b, b_hh_l1b,
        lin_W, lin_b, feat_W, feat_b, v, feature_embeddings,
    )


# speedup vs baseline: 1.0962x; 1.0131x over previous
import functools

import jax
import jax.numpy as jnp
from jax import lax
from jax.experimental import pallas as pl
from jax.experimental.pallas import tpu as pltpu
from jax.experimental.pallas import tpu_sc as plsc

B = 16
L = 128
V = 32000
E = 512
H = 512
NF = 256
FR = 192
FE = 128
P = NF - FR

TW = 16
NBLK = L // TW


def _dotT(a, b):
    return lax.dot_general(
        a, b, (((1,), (1,)), ((), ())), preferred_element_type=jnp.float32
    )


def _gather_rows_sc(table, idx):
    n = idx.shape[0]
    d = table.shape[1]
    mesh = plsc.VectorSubcoreMesh(core_axis_name="c", subcore_axis_name="s")
    nw = mesh.num_cores * mesh.num_subcores
    b_per_w = n // nw

    @functools.partial(
        pl.kernel,
        out_type=jax.ShapeDtypeStruct((n, d), jnp.float32),
        mesh=mesh,
        scratch_types=[
            pltpu.VMEM((b_per_w,), jnp.int32),
            pltpu.VMEM((b_per_w, d), jnp.float32),
            pltpu.SemaphoreType.DMA,
        ],
    )
    def gather_kernel(table_hbm, idx_hbm, out_hbm, idx_v, rows_v, sem):
        wid = lax.axis_index("s") * mesh.num_cores + lax.axis_index("c")
        base = wid * b_per_w
        pltpu.sync_copy(idx_hbm.at[pl.ds(base, b_per_w)], idx_v)
        pltpu.async_copy(table_hbm.at[idx_v], rows_v, sem).wait()
        pltpu.sync_copy(rows_v, out_hbm.at[pl.ds(base, b_per_w)])

    return gather_kernel(table, idx)


def _gi0_kernel(x_ref, wf_ref, wb_ref, bf_ref, bb_ref, gf_ref, gb_ref):
    x = x_ref[...].astype(jnp.bfloat16)
    rows = x.shape[0]
    gf = _dotT(x, wf_ref[...]) + bf_ref[...]
    gb = _dotT(x, wb_ref[...]) + bb_ref[...]
    gf_ref[...] = gf.reshape(rows // B, B, 3 * H).astype(jnp.bfloat16)
    gb_ref[...] = gb.reshape(rows // B, B, 3 * H).astype(jnp.bfloat16)


def _gi0_proj(x_flat, w_f, w_b, b_f, b_b):
    tl = 64
    rows = tl * B
    out_shape = jax.ShapeDtypeStruct((L, B, 3 * H), jnp.bfloat16)
    return pl.pallas_call(
        _gi0_kernel,
        grid=(L // tl,),
        in_specs=[
            pl.BlockSpec((rows, E), lambda i: (i, 0)),
            pl.BlockSpec((3 * H, E), lambda i: (0, 0)),
            pl.BlockSpec((3 * H, E), lambda i: (0, 0)),
            pl.BlockSpec((1, 3 * H), lambda i: (0, 0)),
            pl.BlockSpec((1, 3 * H), lambda i: (0, 0)),
        ],
        out_specs=[
            pl.BlockSpec((tl, B, 3 * H), lambda i: (i, 0, 0)),
            pl.BlockSpec((tl, B, 3 * H), lambda i: (i, 0, 0)),
        ],
        out_shape=[out_shape, out_shape],
    )(x_flat, w_f, w_b, b_f.reshape(1, 3 * H), b_b.reshape(1, 3 * H))


def _gi1_kernel(yf_ref, yb_ref, wf_ref, wb_ref, bf_ref, bb_ref, gf_ref, gb_ref):
    yf = yf_ref[...]
    yb = yb_ref[...]
    rows = yf.shape[0]
    wf = wf_ref[...]
    wb = wb_ref[...]
    gf = _dotT(yf, wf[:, :H]) + _dotT(yb, wf[:, H:]) + bf_ref[...]
    gb = _dotT(yf, wb[:, :H]) + _dotT(yb, wb[:, H:]) + bb_ref[...]
    gf_ref[...] = gf.reshape(rows // B, B, 3 * H).astype(jnp.bfloat16)
    gb_ref[...] = gb.reshape(rows // B, B, 3 * H).astype(jnp.bfloat16)


def _gi1_proj(yf_flat, yb_flat, w_f, w_b, b_f, b_b):
    tl = 64
    rows = tl * B
    out_shape = jax.ShapeDtypeStruct((L, B, 3 * H), jnp.bfloat16)
    return pl.pallas_call(
        _gi1_kernel,
        grid=(L // tl,),
        in_specs=[
            pl.BlockSpec((rows, H), lambda i: (i, 0)),
            pl.BlockSpec((rows, H), lambda i: (i, 0)),
            pl.BlockSpec((3 * H, 2 * H), lambda i: (0, 0)),
            pl.BlockSpec((3 * H, 2 * H), lambda i: (0, 0)),
            pl.BlockSpec((1, 3 * H), lambda i: (0, 0)),
            pl.BlockSpec((1, 3 * H), lambda i: (0, 0)),
        ],
        out_specs=[
            pl.BlockSpec((tl, B, 3 * H), lambda i: (i, 0, 0)),
            pl.BlockSpec((tl, B, 3 * H), lambda i: (i, 0, 0)),
        ],
        out_shape=[out_shape, out_shape],
    )(yf_flat, yb_flat, w_f, w_b, b_f.reshape(1, 3 * H), b_b.reshape(1, 3 * H))


def _gru_cell(gi, gh, h):
    ir, iz, inn = gi[:, :H], gi[:, H : 2 * H], gi[:, 2 * H :]
    hr, hz, hn = gh[:, :H], gh[:, H : 2 * H], gh[:, 2 * H :]
    r = jax.nn.sigmoid(ir + hr)
    z = jax.nn.sigmoid(iz + hz)
    n = jnp.tanh(inn + r * hn)
    return (1.0 - z) * n + z * h


def _cell_step(gi, h, wh, bh):
    gh = lax.dot_general(
        h.astype(jnp.bfloat16), wh, (((1,), (0,)), ((), ())),
        preferred_element_type=jnp.float32,
    ) + bh
    return _gru_cell(gi, gh, h)


def _gru_scan_kernel_t(
    gif_ref, gib_ref, whf_ref, whb_ref, bhf_ref, bhb_ref, h0f_ref, h0b_ref,
    yf_ref, yb_ref, hf, hb, wtf, wtb,
):
    i = pl.program_id(0)

    @pl.when(i == 0)
    def _():
        hf[...] = h0f_ref[...]
        hb[...] = h0b_ref[...]

    @pl.when(i == 0)
    def _():
        wtf[...] = whf_ref[...].T.astype(jnp.bfloat16)
        wtb[...] = whb_ref[...].T.astype(jnp.bfloat16)

    whf = wtf[...]
    whb = wtb[...]
    bhf = bhf_ref[...]
    bhb = bhb_ref[...]
    h_f = hf[...]
    h_b = hb[...]
    for k in range(TW):
        h_f = _cell_step(gif_ref[k], h_f, whf, bhf)
        yf_ref[k] = h_f.astype(jnp.bfloat16)
        h_b = _cell_step(gib_ref[TW - 1 - k], h_b, whb, bhb)
        yb_ref[TW - 1 - k] = h_b.astype(jnp.bfloat16)
    hf[...] = h_f
    hb[...] = h_b


def _gru_scan_kernel_b(
    gif_ref, gib_ref, whf_ref, whb_ref, bhf_ref, bhb_ref, h0f_ref, h0b_ref,
    yf_ref, yb_ref, hf, hb, wtf, wtb,
):
    i = pl.program_id(0)

    @pl.when(i == 0)
    def _():
        hf[...] = h0f_ref[...]
        hb[...] = h0b_ref[...]

    @pl.when(i == 0)
    def _():
        wtf[...] = whf_ref[...].T.astype(jnp.bfloat16)
        wtb[...] = whb_ref[...].T.astype(jnp.bfloat16)

    whf = wtf[...]
    whb = wtb[...]
    bhf = bhf_ref[...]
    bhb = bhb_ref[...]
    h_f = hf[...]
    h_b = hb[...]
    for k in range(TW):
        h_f = _cell_step(gif_ref[k], h_f, whf, bhf)
        yf_ref[:, k, :] = h_f
        h_b = _cell_step(gib_ref[TW - 1 - k], h_b, whb, bhb)
        yb_ref[:, TW - 1 - k, :] = h_b
    hf[...] = h_f
    hb[...] = h_b


def _gru_bidir(gi_f, gi_b, w_hh_f, w_hh_b, b_hh_f, b_hh_b, h0f, h0b, bmajor):
    if bmajor:
        body = _gru_scan_kernel_b
        out_shape = jax.ShapeDtypeStruct((B, L, H), jnp.float32)
        out_specs = [
            pl.BlockSpec((B, TW, H), lambda i: (0, i, 0)),
            pl.BlockSpec((B, TW, H), lambda i: (0, NBLK - 1 - i, 0)),
        ]
    else:
        body = _gru_scan_kernel_t
        out_shape = jax.ShapeDtypeStruct((L, B, H), jnp.bfloat16)
        out_specs = [
            pl.BlockSpec((TW, B, H), lambda i: (i, 0, 0)),
            pl.BlockSpec((TW, B, H), lambda i: (NBLK - 1 - i, 0, 0)),
        ]
    return pl.pallas_call(
        body,
        grid=(NBLK,),
        in_specs=[
            pl.BlockSpec((TW, B, 3 * H), lambda i: (i, 0, 0)),
            pl.BlockSpec((TW, B, 3 * H), lambda i: (NBLK - 1 - i, 0, 0)),
            pl.BlockSpec((3 * H, H), lambda i: (0, 0)),
            pl.BlockSpec((3 * H, H), lambda i: (0, 0)),
            pl.BlockSpec((1, 3 * H), lambda i: (0, 0)),
            pl.BlockSpec((1, 3 * H), lambda i: (0, 0)),
            pl.BlockSpec((B, H), lambda i: (0, 0)),
            pl.BlockSpec((B, H), lambda i: (0, 0)),
        ],
        out_specs=out_specs,
        out_shape=[out_shape, out_shape],
        scratch_shapes=[
            pltpu.VMEM((B, H), jnp.float32),
            pltpu.VMEM((B, H), jnp.float32),
            pltpu.VMEM((H, 3 * H), jnp.bfloat16),
            pltpu.VMEM((H, 3 * H), jnp.bfloat16),
        ],
    )(
        gi_f, gi_b,
        w_hh_f, w_hh_b,
        b_hh_f.reshape(1, 3 * H), b_hh_b.reshape(1, 3 * H), h0f, h0b,
    )


def _gru_scan_l1_kernel(
    yfi_ref, ybi_ref, yfr_ref, ybr_ref, w1f_ref, w1b_ref, b1f_ref, b1b_ref,
    whf_ref, whb_ref, bhf_ref, bhb_ref, h0f_ref, h0b_ref,
    yf_ref, yb_ref, hf, hb, wtf, wtb,
):
    i = pl.program_id(0)

    @pl.when(i == 0)
    def _():
        hf[...] = h0f_ref[...]
        hb[...] = h0b_ref[...]

    @pl.when(i == 0)
    def _():
        wtf[...] = whf_ref[...].T.astype(jnp.bfloat16)
        wtb[...] = whb_ref[...].T.astype(jnp.bfloat16)

    w1f = w1f_ref[...]
    w1b = w1b_ref[...]
    gif = (
        _dotT(yfi_ref[...].reshape(TW * B, H), w1f[:, :H])
        + _dotT(ybi_ref[...].reshape(TW * B, H), w1f[:, H:])
        + b1f_ref[...]
    ).reshape(TW, B, 3 * H)
    gib = (
        _dotT(yfr_ref[...].reshape(TW * B, H), w1b[:, :H])
        + _dotT(ybr_ref[...].reshape(TW * B, H), w1b[:, H:])
        + b1b_ref[...]
    ).reshape(TW, B, 3 * H)

    whf = wtf[...]
    whb = wtb[...]
    bhf = bhf_ref[...]
    bhb = bhb_ref[...]
    h_f = hf[...]
    h_b = hb[...]
    for k in range(TW):
        h_f = _cell_step(gif[k], h_f, whf, bhf)
        yf_ref[:, k, :] = h_f
        h_b = _cell_step(gib[TW - 1 - k], h_b, whb, bhb)
        yb_ref[:, TW - 1 - k, :] = h_b
    hf[...] = h_f
    hb[...] = h_b


def _gru_l1_fused(yf, yb, w1f, w1b, b1f, b1b, w_hh_f, w_hh_b, b_hh_f, b_hh_b,
                  h0f, h0b):
    out_shape = jax.ShapeDtypeStruct((B, L, H), jnp.float32)
    ymap_f = lambda i: (i, 0, 0)
    ymap_r = lambda i: (NBLK - 1 - i, 0, 0)
    const2 = lambda shape: pl.BlockSpec(shape, lambda i: (0, 0))
    return pl.pallas_call(
        _gru_scan_l1_kernel,
        grid=(NBLK,),
        in_specs=[
            pl.BlockSpec((TW, B, H), ymap_f),
            pl.BlockSpec((TW, B, H), ymap_f),
            pl.BlockSpec((TW, B, H), ymap_r),
            pl.BlockSpec((TW, B, H), ymap_r),
            const2((3 * H, 2 * H)),
            const2((3 * H, 2 * H)),
            const2((1, 3 * H)),
            const2((1, 3 * H)),
            const2((3 * H, H)),
            const2((3 * H, H)),
            const2((1, 3 * H)),
            const2((1, 3 * H)),
            const2((B, H)),
            const2((B, H)),
        ],
        out_specs=[
            pl.BlockSpec((B, TW, H), lambda i: (0, i, 0)),
            pl.BlockSpec((B, TW, H), lambda i: (0, NBLK - 1 - i, 0)),
        ],
        out_shape=[out_shape, out_shape],
        scratch_shapes=[
            pltpu.VMEM((B, H), jnp.float32),
            pltpu.VMEM((B, H), jnp.float32),
            pltpu.VMEM((H, 3 * H), jnp.bfloat16),
            pltpu.VMEM((H, 3 * H), jnp.bfloat16),
        ],
    )(
        yf, yb, yf, yb,
        w1f.astype(jnp.bfloat16), w1b.astype(jnp.bfloat16),
        b1f.reshape(1, 3 * H), b1b.reshape(1, 3 * H),
        w_hh_f, w_hh_b,
        b_hh_f.reshape(1, 3 * H), b_hh_b.reshape(1, 3 * H), h0f, h0b,
    )


def _attn_kernel(
    yf_ref, yb_ref, fe_ref, fw_ref, fb_ref, v_ref, lw_ref, lb_ref,
    ctx_ref, attn_ref,
):
    fe = fe_ref[...]
    vv = v_ref[...]
    mf = jnp.dot(fe, fw_ref[...], preferred_element_type=jnp.float32)
    mv = jnp.dot(vv, lw_ref[...], preferred_element_type=jnp.float32)
    m_mat = jnp.concatenate([mf, mv], axis=0)
    cf = _dotT(fb_ref[...], fe)
    cv = _dotT(lb_ref[...], vv)
    c = jnp.concatenate([cf, cv], axis=1)
    yf = yf_ref[...]
    yb = yb_ref[...]
    energy = (
        _dotT(yf.reshape(B * L, H), m_mat[:, :H])
        + _dotT(yb.reshape(B * L, H), m_mat[:, H:])
        + c
    )
    e3 = energy.reshape(B, L, NF)
    mx = jnp.max(e3, axis=0, keepdims=True)
    ex = jnp.exp(e3 - mx)
    sm = jnp.sum(ex, axis=0, keepdims=True)
    soft3 = ex / sm
    for b in range(B):
        s = soft3[b]
        ctx_ref[b, :, :H] = lax.dot_general(
            s, yf[b], (((0,), (0,)), ((), ())),
            preferred_element_type=jnp.float32,
        )
        ctx_ref[b, :, H:] = lax.dot_general(
            s, yb[b], (((0,), (0,)), ((), ())),
            preferred_element_type=jnp.float32,
        )
        attn_ref[b] = s.T


def _attention(yf2, yb2, feature_embeddings, feat_W, feat_b, v, lin_W, lin_b):
    return pl.pallas_call(
        _attn_kernel,
        out_shape=[
            jax.ShapeDtypeStruct((B, NF, 2 * H), jnp.float32),
            jax.ShapeDtypeStruct((B, NF, L), jnp.float32),
        ],
    )(
        yf2, yb2, feature_embeddings, feat_W, feat_b.reshape(1, FE),
        v, lin_W, lin_b.reshape(1, H),
    )


def _encode_tc(
    xs_flat, hidden,
    w_ih_l0f, w_hh_l0f, b_ih_l0f, b_hh_l0f,
    w_ih_l0b, w_hh_l0b, b_ih_l0b, b_hh_l0b,
    w_ih_l1f, w_hh_l1f, b_ih_l1f, b_hh_l1f,
    w_ih_l1b, w_hh_l1b, b_ih_l1b, b_hh_l1b,
    lin_W, lin_b, feat_W, feat_b, v, feature_embeddings,
):
    gi0f, gi0b = _gi0_proj(
        xs_flat,
        w_ih_l0f.astype(jnp.bfloat16), w_ih_l0b.astype(jnp.bfloat16),
        b_ih_l0f, b_ih_l0b,
    )
    yf, yb = _gru_bidir(
        gi0f, gi0b, w_hh_l0f, w_hh_l0b, b_hh_l0f, b_hh_l0b,
        hidden[0], hidden[1], bmajor=False,
    )
    yf2, yb2 = _gru_l1_fused(
        yf, yb, w_ih_l1f, w_ih_l1b, b_ih_l1f, b_ih_l1b,
        w_hh_l1f, w_hh_l1b, b_hh_l1f, b_hh_l1b,
        hidden[2], hidden[3],
    )
    context, attn_w = _attention(
        yf2, yb2, feature_embeddings, feat_W, feat_b, v, lin_W, lin_b,
    )
    return context, attn_w


def kernel(
    input_variable, input_lengths, hidden, feature_embeddings, emb_table,
    w_ih_l0f, w_hh_l0f, b_ih_l0f, b_hh_l0f,
    w_ih_l0b, w_hh_l0b, b_ih_l0b, b_hh_l0b,
    w_ih_l1f, w_hh_l1f, b_ih_l1f, b_hh_l1f,
    w_ih_l1b, w_hh_l1b, b_ih_l1b, b_hh_l1b,
    lin_W, lin_b, feat_W, feat_b, v,
):
    idx_t = jnp.transpose(input_variable, (1, 0)).reshape(L * B).astype(jnp.int32)
    xs_flat = _gather_rows_sc(emb_table, idx_t)
    return _encode_tc(
        xs_flat, hidden,
        w_ih_l0f, w_hh_l0f, b_ih_l0f, b_hh_l0f,
        w_ih_l0b, w_hh_l0b, b_ih_l0b, b_hh_l0b,
        w_ih_l1f, w_hh_l1f, b_ih_l1f, b_hh_l1f,
        w_ih_l1b, w_hh_l1b, b_ih_l1b, b_hh_l1b,
        lin_W, lin_b, feat_W, feat_b, v, feature_embeddings,
    )


# speedup vs baseline: 1.1102x; 1.0129x over previous
import functools

import jax
import jax.numpy as jnp
from jax import lax
from jax.experimental import pallas as pl
from jax.experimental.pallas import tpu as pltpu
from jax.experimental.pallas import tpu_sc as plsc

B = 16
L = 128
V = 32000
E = 512
H = 512
NF = 256
FR = 192
FE = 128
P = NF - FR

TW = 16
NBLK = L // TW


def _dotT(a, b):
    return lax.dot_general(
        a, b, (((1,), (1,)), ((), ())), preferred_element_type=jnp.float32
    )


def _gather_rows_sc(table, idx):
    n = idx.shape[0]
    d = table.shape[1]
    mesh = plsc.VectorSubcoreMesh(core_axis_name="c", subcore_axis_name="s")
    nw = mesh.num_cores * mesh.num_subcores
    b_per_w = n // nw

    @functools.partial(
        pl.kernel,
        out_type=jax.ShapeDtypeStruct((n, d), jnp.float32),
        mesh=mesh,
        scratch_types=[
            pltpu.VMEM((b_per_w,), jnp.int32),
            pltpu.VMEM((b_per_w, d), jnp.float32),
            pltpu.SemaphoreType.DMA,
        ],
    )
    def gather_kernel(table_hbm, idx_hbm, out_hbm, idx_v, rows_v, sem):
        wid = lax.axis_index("s") * mesh.num_cores + lax.axis_index("c")
        base = wid * b_per_w
        pltpu.sync_copy(idx_hbm.at[pl.ds(base, b_per_w)], idx_v)
        pltpu.async_copy(table_hbm.at[idx_v], rows_v, sem).wait()
        pltpu.sync_copy(rows_v, out_hbm.at[pl.ds(base, b_per_w)])

    return gather_kernel(table, idx)


def _gi0_kernel(x_ref, wf_ref, wb_ref, bf_ref, bb_ref, gf_ref, gb_ref):
    x = x_ref[...].astype(jnp.bfloat16)
    rows = x.shape[0]
    gf = _dotT(x, wf_ref[...]) + bf_ref[...]
    gb = _dotT(x, wb_ref[...]) + bb_ref[...]
    gf_ref[...] = gf.reshape(rows // B, B, 3 * H).astype(jnp.bfloat16)
    gb_ref[...] = gb.reshape(rows // B, B, 3 * H).astype(jnp.bfloat16)


def _gi0_proj(x_flat, w_f, w_b, b_f, b_b):
    tl = 64
    rows = tl * B
    out_shape = jax.ShapeDtypeStruct((L, B, 3 * H), jnp.bfloat16)
    return pl.pallas_call(
        _gi0_kernel,
        grid=(L // tl,),
        in_specs=[
            pl.BlockSpec((rows, E), lambda i: (i, 0)),
            pl.BlockSpec((3 * H, E), lambda i: (0, 0)),
            pl.BlockSpec((3 * H, E), lambda i: (0, 0)),
            pl.BlockSpec((1, 3 * H), lambda i: (0, 0)),
            pl.BlockSpec((1, 3 * H), lambda i: (0, 0)),
        ],
        out_specs=[
            pl.BlockSpec((tl, B, 3 * H), lambda i: (i, 0, 0)),
            pl.BlockSpec((tl, B, 3 * H), lambda i: (i, 0, 0)),
        ],
        out_shape=[out_shape, out_shape],
    )(x_flat, w_f, w_b, b_f.reshape(1, 3 * H), b_b.reshape(1, 3 * H))


def _gi1_kernel(yf_ref, yb_ref, wf_ref, wb_ref, bf_ref, bb_ref, gf_ref, gb_ref):
    yf = yf_ref[...]
    yb = yb_ref[...]
    rows = yf.shape[0]
    wf = wf_ref[...]
    wb = wb_ref[...]
    gf = _dotT(yf, wf[:, :H]) + _dotT(yb, wf[:, H:]) + bf_ref[...]
    gb = _dotT(yf, wb[:, :H]) + _dotT(yb, wb[:, H:]) + bb_ref[...]
    gf_ref[...] = gf.reshape(rows // B, B, 3 * H).astype(jnp.bfloat16)
    gb_ref[...] = gb.reshape(rows // B, B, 3 * H).astype(jnp.bfloat16)


def _gi1_proj(yf_flat, yb_flat, w_f, w_b, b_f, b_b):
    tl = 64
    rows = tl * B
    out_shape = jax.ShapeDtypeStruct((L, B, 3 * H), jnp.bfloat16)
    return pl.pallas_call(
        _gi1_kernel,
        grid=(L // tl,),
        in_specs=[
            pl.BlockSpec((rows, H), lambda i: (i, 0)),
            pl.BlockSpec((rows, H), lambda i: (i, 0)),
            pl.BlockSpec((3 * H, 2 * H), lambda i: (0, 0)),
            pl.BlockSpec((3 * H, 2 * H), lambda i: (0, 0)),
            pl.BlockSpec((1, 3 * H), lambda i: (0, 0)),
            pl.BlockSpec((1, 3 * H), lambda i: (0, 0)),
        ],
        out_specs=[
            pl.BlockSpec((tl, B, 3 * H), lambda i: (i, 0, 0)),
            pl.BlockSpec((tl, B, 3 * H), lambda i: (i, 0, 0)),
        ],
        out_shape=[out_shape, out_shape],
    )(yf_flat, yb_flat, w_f, w_b, b_f.reshape(1, 3 * H), b_b.reshape(1, 3 * H))


def _gru_cell(gi, gh, h):
    ir, iz, inn = gi[:, :H], gi[:, H : 2 * H], gi[:, 2 * H :]
    hr, hz, hn = gh[:, :H], gh[:, H : 2 * H], gh[:, 2 * H :]
    r = jax.nn.sigmoid(ir + hr)
    z = jax.nn.sigmoid(iz + hz)
    n = jnp.tanh(inn + r * hn)
    return (1.0 - z) * n + z * h


def _cell_step(gi, h, wh, bh):
    gh = lax.dot_general(
        h.astype(jnp.bfloat16), wh, (((1,), (0,)), ((), ())),
        preferred_element_type=jnp.float32,
    ) + bh
    return _gru_cell(gi, gh, h)


def _gru_scan_kernel_t(
    gif_ref, gib_ref, whf_ref, whb_ref, bhf_ref, bhb_ref, h0f_ref, h0b_ref,
    yf_ref, yb_ref, hf, hb, wtf, wtb,
):
    i = pl.program_id(0)

    @pl.when(i == 0)
    def _():
        hf[...] = h0f_ref[...]
        hb[...] = h0b_ref[...]

    @pl.when(i == 0)
    def _():
        wtf[...] = whf_ref[...].T.astype(jnp.bfloat16)
        wtb[...] = whb_ref[...].T.astype(jnp.bfloat16)

    whf = wtf[...]
    whb = wtb[...]
    bhf = bhf_ref[...]
    bhb = bhb_ref[...]
    h_f = hf[...]
    h_b = hb[...]
    for k in range(TW):
        h_f = _cell_step(gif_ref[k], h_f, whf, bhf)
        yf_ref[k] = h_f.astype(jnp.bfloat16)
        h_b = _cell_step(gib_ref[TW - 1 - k], h_b, whb, bhb)
        yb_ref[TW - 1 - k] = h_b.astype(jnp.bfloat16)
    hf[...] = h_f
    hb[...] = h_b


def _gru_scan_kernel_b(
    gif_ref, gib_ref, whf_ref, whb_ref, bhf_ref, bhb_ref, h0f_ref, h0b_ref,
    yf_ref, yb_ref, hf, hb, wtf, wtb,
):
    i = pl.program_id(0)

    @pl.when(i == 0)
    def _():
        hf[...] = h0f_ref[...]
        hb[...] = h0b_ref[...]

    @pl.when(i == 0)
    def _():
        wtf[...] = whf_ref[...].T.astype(jnp.bfloat16)
        wtb[...] = whb_ref[...].T.astype(jnp.bfloat16)

    whf = wtf[...]
    whb = wtb[...]
    bhf = bhf_ref[...]
    bhb = bhb_ref[...]
    h_f = hf[...]
    h_b = hb[...]
    for k in range(TW):
        h_f = _cell_step(gif_ref[k], h_f, whf, bhf)
        yf_ref[:, k, :] = h_f
        h_b = _cell_step(gib_ref[TW - 1 - k], h_b, whb, bhb)
        yb_ref[:, TW - 1 - k, :] = h_b
    hf[...] = h_f
    hb[...] = h_b


def _gru_bidir(gi_f, gi_b, w_hh_f, w_hh_b, b_hh_f, b_hh_b, h0f, h0b, bmajor):
    if bmajor:
        body = _gru_scan_kernel_b
        out_shape = jax.ShapeDtypeStruct((B, L, H), jnp.float32)
        out_specs = [
            pl.BlockSpec((B, TW, H), lambda i: (0, i, 0)),
            pl.BlockSpec((B, TW, H), lambda i: (0, NBLK - 1 - i, 0)),
        ]
    else:
        body = _gru_scan_kernel_t
        out_shape = jax.ShapeDtypeStruct((L, B, H), jnp.bfloat16)
        out_specs = [
            pl.BlockSpec((TW, B, H), lambda i: (i, 0, 0)),
            pl.BlockSpec((TW, B, H), lambda i: (NBLK - 1 - i, 0, 0)),
        ]
    return pl.pallas_call(
        body,
        grid=(NBLK,),
        in_specs=[
            pl.BlockSpec((TW, B, 3 * H), lambda i: (i, 0, 0)),
            pl.BlockSpec((TW, B, 3 * H), lambda i: (NBLK - 1 - i, 0, 0)),
            pl.BlockSpec((3 * H, H), lambda i: (0, 0)),
            pl.BlockSpec((3 * H, H), lambda i: (0, 0)),
            pl.BlockSpec((1, 3 * H), lambda i: (0, 0)),
            pl.BlockSpec((1, 3 * H), lambda i: (0, 0)),
            pl.BlockSpec((B, H), lambda i: (0, 0)),
            pl.BlockSpec((B, H), lambda i: (0, 0)),
        ],
        out_specs=out_specs,
        out_shape=[out_shape, out_shape],
        scratch_shapes=[
            pltpu.VMEM((B, H), jnp.float32),
            pltpu.VMEM((B, H), jnp.float32),
            pltpu.VMEM((H, 3 * H), jnp.bfloat16),
            pltpu.VMEM((H, 3 * H), jnp.bfloat16),
        ],
    )(
        gi_f, gi_b,
        w_hh_f, w_hh_b,
        b_hh_f.reshape(1, 3 * H), b_hh_b.reshape(1, 3 * H), h0f, h0b,
    )


def _gru_scan_l0_kernel(
    xi_ref, xr_ref, w0f_ref, w0b_ref, b0f_ref, b0b_ref,
    whf_ref, whb_ref, bhf_ref, bhb_ref, h0f_ref, h0b_ref,
    yf_ref, yb_ref, hf, hb, wtf, wtb,
):
    i = pl.program_id(0)

    @pl.when(i == 0)
    def _():
        hf[...] = h0f_ref[...]
        hb[...] = h0b_ref[...]

    @pl.when(i == 0)
    def _():
        wtf[...] = whf_ref[...].T.astype(jnp.bfloat16)
        wtb[...] = whb_ref[...].T.astype(jnp.bfloat16)

    gif = (
        _dotT(xi_ref[...].reshape(TW * B, E).astype(jnp.bfloat16), w0f_ref[...])
        + b0f_ref[...]
    ).reshape(TW, B, 3 * H)
    gib = (
        _dotT(xr_ref[...].reshape(TW * B, E).astype(jnp.bfloat16), w0b_ref[...])
        + b0b_ref[...]
    ).reshape(TW, B, 3 * H)

    whf = wtf[...]
    whb = wtb[...]
    bhf = bhf_ref[...]
    bhb = bhb_ref[...]
    h_f = hf[...]
    h_b = hb[...]
    for k in range(TW):
        h_f = _cell_step(gif[k], h_f, whf, bhf)
        yf_ref[k] = h_f.astype(jnp.bfloat16)
        h_b = _cell_step(gib[TW - 1 - k], h_b, whb, bhb)
        yb_ref[TW - 1 - k] = h_b.astype(jnp.bfloat16)
    hf[...] = h_f
    hb[...] = h_b


def _gru_l0_fused(x, w0f, w0b, b0f, b0b, w_hh_f, w_hh_b, b_hh_f, b_hh_b,
                  h0f, h0b):
    out_shape = jax.ShapeDtypeStruct((L, B, H), jnp.bfloat16)
    const2 = lambda shape: pl.BlockSpec(shape, lambda i: (0, 0))
    return pl.pallas_call(
        _gru_scan_l0_kernel,
        grid=(NBLK,),
        in_specs=[
            pl.BlockSpec((TW, B, E), lambda i: (i, 0, 0)),
            pl.BlockSpec((TW, B, E), lambda i: (NBLK - 1 - i, 0, 0)),
            const2((3 * H, E)),
            const2((3 * H, E)),
            const2((1, 3 * H)),
            const2((1, 3 * H)),
            const2((3 * H, H)),
            const2((3 * H, H)),
            const2((1, 3 * H)),
            const2((1, 3 * H)),
            const2((B, H)),
            const2((B, H)),
        ],
        out_specs=[
            pl.BlockSpec((TW, B, H), lambda i: (i, 0, 0)),
            pl.BlockSpec((TW, B, H), lambda i: (NBLK - 1 - i, 0, 0)),
        ],
        out_shape=[out_shape, out_shape],
        scratch_shapes=[
            pltpu.VMEM((B, H), jnp.float32),
            pltpu.VMEM((B, H), jnp.float32),
            pltpu.VMEM((H, 3 * H), jnp.bfloat16),
            pltpu.VMEM((H, 3 * H), jnp.bfloat16),
        ],
    )(
        x, x,
        w0f.astype(jnp.bfloat16), w0b.astype(jnp.bfloat16),
        b0f.reshape(1, 3 * H), b0b.reshape(1, 3 * H),
        w_hh_f, w_hh_b,
        b_hh_f.reshape(1, 3 * H), b_hh_b.reshape(1, 3 * H), h0f, h0b,
    )


def _gru_scan_l1_kernel(
    yfi_ref, ybi_ref, yfr_ref, ybr_ref, w1f_ref, w1b_ref, b1f_ref, b1b_ref,
    whf_ref, whb_ref, bhf_ref, bhb_ref, h0f_ref, h0b_ref,
    yf_ref, yb_ref, hf, hb, wtf, wtb,
):
    i = pl.program_id(0)

    @pl.when(i == 0)
    def _():
        hf[...] = h0f_ref[...]
        hb[...] = h0b_ref[...]

    @pl.when(i == 0)
    def _():
        wtf[...] = whf_ref[...].T.astype(jnp.bfloat16)
        wtb[...] = whb_ref[...].T.astype(jnp.bfloat16)

    w1f = w1f_ref[...]
    w1b = w1b_ref[...]
    gif = (
        _dotT(yfi_ref[...].reshape(TW * B, H), w1f[:, :H])
        + _dotT(ybi_ref[...].reshape(TW * B, H), w1f[:, H:])
        + b1f_ref[...]
    ).reshape(TW, B, 3 * H)
    gib = (
        _dotT(yfr_ref[...].reshape(TW * B, H), w1b[:, :H])
        + _dotT(ybr_ref[...].reshape(TW * B, H), w1b[:, H:])
        + b1b_ref[...]
    ).reshape(TW, B, 3 * H)

    whf = wtf[...]
    whb = wtb[...]
    bhf = bhf_ref[...]
    bhb = bhb_ref[...]
    h_f = hf[...]
    h_b = hb[...]
    for k in range(TW):
        h_f = _cell_step(gif[k], h_f, whf, bhf)
        yf_ref[:, k, :] = h_f
        h_b = _cell_step(gib[TW - 1 - k], h_b, whb, bhb)
        yb_ref[:, TW - 1 - k, :] = h_b
    hf[...] = h_f
    hb[...] = h_b


def _gru_l1_fused(yf, yb, w1f, w1b, b1f, b1b, w_hh_f, w_hh_b, b_hh_f, b_hh_b,
                  h0f, h0b):
    out_shape = jax.ShapeDtypeStruct((B, L, H), jnp.float32)
    ymap_f = lambda i: (i, 0, 0)
    ymap_r = lambda i: (NBLK - 1 - i, 0, 0)
    const2 = lambda shape: pl.BlockSpec(shape, lambda i: (0, 0))
    return pl.pallas_call(
        _gru_scan_l1_kernel,
        grid=(NBLK,),
        in_specs=[
            pl.BlockSpec((TW, B, H), ymap_f),
            pl.BlockSpec((TW, B, H), ymap_f),
            pl.BlockSpec((TW, B, H), ymap_r),
            pl.BlockSpec((TW, B, H), ymap_r),
            const2((3 * H, 2 * H)),
            const2((3 * H, 2 * H)),
            const2((1, 3 * H)),
            const2((1, 3 * H)),
            const2((3 * H, H)),
            const2((3 * H, H)),
            const2((1, 3 * H)),
            const2((1, 3 * H)),
            const2((B, H)),
            const2((B, H)),
        ],
        out_specs=[
            pl.BlockSpec((B, TW, H), lambda i: (0, i, 0)),
            pl.BlockSpec((B, TW, H), lambda i: (0, NBLK - 1 - i, 0)),
        ],
        out_shape=[out_shape, out_shape],
        scratch_shapes=[
            pltpu.VMEM((B, H), jnp.float32),
            pltpu.VMEM((B, H), jnp.float32),
            pltpu.VMEM((H, 3 * H), jnp.bfloat16),
            pltpu.VMEM((H, 3 * H), jnp.bfloat16),
        ],
    )(
        yf, yb, yf, yb,
        w1f.astype(jnp.bfloat16), w1b.astype(jnp.bfloat16),
        b1f.reshape(1, 3 * H), b1b.reshape(1, 3 * H),
        w_hh_f, w_hh_b,
        b_hh_f.reshape(1, 3 * H), b_hh_b.reshape(1, 3 * H), h0f, h0b,
    )


def _attn_kernel(
    yf_ref, yb_ref, fe_ref, fw_ref, fb_ref, v_ref, lw_ref, lb_ref,
    ctx_ref, attn_ref,
):
    fe = fe_ref[...]
    vv = v_ref[...]
    mf = jnp.dot(fe, fw_ref[...], preferred_element_type=jnp.float32)
    mv = jnp.dot(vv, lw_ref[...], preferred_element_type=jnp.float32)
    m_mat = jnp.concatenate([mf, mv], axis=0)
    cf = _dotT(fb_ref[...], fe)
    cv = _dotT(lb_ref[...], vv)
    c = jnp.concatenate([cf, cv], axis=1)
    yf = yf_ref[...]
    yb = yb_ref[...]
    energy = (
        _dotT(yf.reshape(B * L, H), m_mat[:, :H])
        + _dotT(yb.reshape(B * L, H), m_mat[:, H:])
        + c
    )
    e3 = energy.reshape(B, L, NF)
    mx = jnp.max(e3, axis=0, keepdims=True)
    ex = jnp.exp(e3 - mx)
    sm = jnp.sum(ex, axis=0, keepdims=True)
    soft3 = ex / sm
    for b in range(B):
        s = soft3[b]
        ctx_ref[b, :, :H] = lax.dot_general(
            s, yf[b], (((0,), (0,)), ((), ())),
            preferred_element_type=jnp.float32,
        )
        ctx_ref[b, :, H:] = lax.dot_general(
            s, yb[b], (((0,), (0,)), ((), ())),
            preferred_element_type=jnp.float32,
        )
        attn_ref[b] = s.T


def _attention(yf2, yb2, feature_embeddings, feat_W, feat_b, v, lin_W, lin_b):
    return pl.pallas_call(
        _attn_kernel,
        out_shape=[
            jax.ShapeDtypeStruct((B, NF, 2 * H), jnp.float32),
            jax.ShapeDtypeStruct((B, NF, L), jnp.float32),
        ],
    )(
        yf2, yb2, feature_embeddings, feat_W, feat_b.reshape(1, FE),
        v, lin_W, lin_b.reshape(1, H),
    )


def _encode_tc(
    xs_flat, hidden,
    w_ih_l0f, w_hh_l0f, b_ih_l0f, b_hh_l0f,
    w_ih_l0b, w_hh_l0b, b_ih_l0b, b_hh_l0b,
    w_ih_l1f, w_hh_l1f, b_ih_l1f, b_hh_l1f,
    w_ih_l1b, w_hh_l1b, b_ih_l1b, b_hh_l1b,
    lin_W, lin_b, feat_W, feat_b, v, feature_embeddings,
):
    yf, yb = _gru_l0_fused(
        xs_flat.reshape(L, B, E),
        w_ih_l0f, w_ih_l0b, b_ih_l0f, b_ih_l0b,
        w_hh_l0f, w_hh_l0b, b_hh_l0f, b_hh_l0b,
        hidden[0], hidden[1],
    )
    yf2, yb2 = _gru_l1_fused(
        yf, yb, w_ih_l1f, w_ih_l1b, b_ih_l1f, b_ih_l1b,
        w_hh_l1f, w_hh_l1b, b_hh_l1f, b_hh_l1b,
        hidden[2], hidden[3],
    )
    context, attn_w = _attention(
        yf2, yb2, feature_embeddings, feat_W, feat_b, v, lin_W, lin_b,
    )
    return context, attn_w


def kernel(
    input_variable, input_lengths, hidden, feature_embeddings, emb_table,
    w_ih_l0f, w_hh_l0f, b_ih_l0f, b_hh_l0f,
    w_ih_l0b, w_hh_l0b, b_ih_l0b, b_hh_l0b,
    w_ih_l1f, w_hh_l1f, b_ih_l1f, b_hh_l1f,
    w_ih_l1b, w_hh_l1b, b_ih_l1b, b_hh_l1b,
    lin_W, lin_b, feat_W, feat_b, v,
):
    idx_t = jnp.transpose(input_variable, (1, 0)).reshape(L * B).astype(jnp.int32)
    xs_flat = _gather_rows_sc(emb_table, idx_t)
    return _encode_tc(
        xs_flat, hidden,
        w_ih_l0f, w_hh_l0f, b_ih_l0f, b_hh_l0f,
        w_ih_l0b, w_hh_l0b, b_ih_l0b, b_hh_l0b,
        w_ih_l1f, w_hh_l1f, b_ih_l1f, b_hh_l1f,
        w_ih_l1b, w_hh_l1b, b_ih_l1b, b_hh_l1b,
        lin_W, lin_b, feat_W, feat_b, v, feature_embeddings,
    )
